# Initial kernel scaffold; baseline (speedup 1.0000x reference)
import jax
import jax.numpy as jnp
from jax.experimental import pallas as pl


def kernel(mel_power):
    raise NotImplementedError("write your pallas kernel here")


# speedup vs baseline: 22.7004x; 22.7004x over previous
import functools

import numpy as np
import jax
import jax.numpy as jnp
from jax.experimental import pallas as pl
from jax.experimental.pallas import tpu as pltpu

_S = 0.025
_ALPHA = 0.98
_DELTA = 2.0
_EPS = 1e-6

_RB = 1024
_L = 128

_IDX = np.arange(_L)
_DIFF = _IDX[None, :] - _IDX[:, None]
_W_NP = np.where(_DIFF >= 0,
                 _S * (1.0 - _S) ** np.maximum(_DIFF, 0),
                 0.0).astype(np.float32)
_D_NP = ((1.0 - _S) ** (_IDX + 1.0)).astype(np.float32)[None, :]
_SQRT_DELTA = float(np.sqrt(_DELTA))


def _pcen_body(x_ref, w_ref, d_ref, o_ref, carry_ref, *, t_total):
    t = pl.program_id(1)

    @pl.when(t == 0)
    def _():
        carry_ref[...] = jnp.zeros_like(carry_ref)

    x = x_ref[...]
    col = jax.lax.broadcasted_iota(jnp.int32, (1, _L), 1)
    x = jnp.where(col < (t_total - t * _L), x, 0.0)

    m = jnp.dot(x, w_ref[...], preferred_element_type=jnp.float32,
                precision=jax.lax.Precision.HIGHEST)
    m = m + carry_ref[...] * d_ref[...]
    carry_ref[...] = m[:, _L - 1:_L]

    p = jnp.exp(-_ALPHA * jnp.log(_EPS + m))
    o_ref[...] = jnp.sqrt(x * p + _DELTA) - _SQRT_DELTA


def kernel(mel_power):
    B, C, T = mel_power.shape
    rows = B * C
    x = mel_power.reshape(rows, T)
    n_r = rows // _RB
    n_t = pl.cdiv(T, _L)
    out = pl.pallas_call(
        functools.partial(_pcen_body, t_total=T),
        grid=(n_r, n_t),
        in_specs=[
            pl.BlockSpec((_RB, _L), lambda r, t: (r, t)),
            pl.BlockSpec((_L, _L), lambda r, t: (0, 0)),
            pl.BlockSpec((1, _L), lambda r, t: (0, 0)),
        ],
        out_specs=pl.BlockSpec((_RB, _L), lambda r, t: (r, t)),
        out_shape=jax.ShapeDtypeStruct((rows, T), jnp.float32),
        scratch_shapes=[pltpu.VMEM((_RB, 1), jnp.float32)],
        compiler_params=pltpu.CompilerParams(
            dimension_semantics=("parallel", "arbitrary"),
        ),
    )(x, jnp.asarray(_W_NP), jnp.asarray(_D_NP))
    return out.reshape(B, C, T)
